# baseline (speedup 1.0000x reference)
import functools

import jax
import jax.numpy as jnp
import numpy as np
from jax import lax
from jax.experimental import pallas as pl
from jax.experimental.pallas import tpu as pltpu
from jax.experimental.pallas import tpu_sc as plsc

_B = 2048
_G = 2048
_N_LEAF = 256
_GPL = 8
_N_MID = 64
_LPM = 4
_GPM = _GPL * _LPM
_EPS = 1e-5


_NW = 32
_MPW = _N_MID // _NW
_L = 16
_OFF_WGR = 2048
_OFF_BGR = 2560
_OFF_WAL = 3072
_WTAB = 3136


def _rsqrt_newton(v):
    i = lax.bitcast_convert_type(v, jnp.int32)
    y = lax.bitcast_convert_type(jnp.int32(0x5F3759DF) - (i >> 1), jnp.float32)
    for _ in range(4):
        y = y * (1.5 - 0.5 * v * y * y)
    return y


def _sc_body(x_hbm, wtab_hbm, out_hbm, xbuf, hbuf, zbuf, wbuf, awbuf,
             idxbuf, sem):
    f32 = jnp.float32
    i32 = jnp.int32
    iota = lax.broadcasted_iota(i32, (_L,), 0)
    z16 = jnp.zeros((_L,), f32)

    def spl(off):
        return wbuf[pl.ds(off * _L, _L)]

    wid = lax.axis_index("s") * 2 + lax.axis_index("c")
    for mm in range(_MPW):
        mid = wid * _MPW + mm
        pltpu.sync_copy(wtab_hbm.at[pl.ds(mid * _WTAB, _WTAB)], wbuf)

        for j in range(16):
            for u in range(8):
                idxbuf[j, pl.ds(u * _L, _L)] = (
                    (j * 128 + u * _L + iota) * _N_MID + mid)
        for j in range(16):
            pltpu.async_copy(x_hbm.at[idxbuf.at[j]],
                             xbuf.at[pl.ds(j * 128, 128)], sem)
        for j in range(16):
            pltpu.make_async_copy(x_hbm.at[idxbuf.at[j]],
                                  xbuf.at[pl.ds(j * 128, 128)], sem).wait()

        def ph_a(ii, carry):
            acc = list(carry)
            for u in range(4):
                row = jnp.full((_L,), ii * 4 + u, i32)
                v0 = plsc.load_gather(xbuf, [row, iota])
                v1 = plsc.load_gather(xbuf, [row, iota + _L])
                acc[0] = acc[0] + v0
                acc[1] = acc[1] + v1
                acc[2] = acc[2] + v0 * v0
                acc[3] = acc[3] + v1 * v1
            return tuple(acc)
        acc = lax.fori_loop(0, _B // 4, ph_a, (z16,) * 4)
        mu_v = [acc[0] * (1.0 / _B), acc[1] * (1.0 / _B)]
        var_v = [acc[2] * (1.0 / _B) - mu_v[0] * mu_v[0],
                 acc[3] * (1.0 / _B) - mu_v[1] * mu_v[1]]
        rinv_v = [_rsqrt_newton(var_v[0] + _EPS),
                  _rsqrt_newton(var_v[1] + _EPS)]
        aw0 = wbuf[pl.ds(_OFF_WAL, _L)] * rinv_v[0]
        aw1 = wbuf[pl.ds(_OFF_WAL + _L, _L)] * rinv_v[1]
        awbuf[pl.ds(0, _L)] = aw0
        awbuf[pl.ds(_L, _L)] = aw1
        lo8 = (iota & 15) < 8
        bc = [jnp.sum(jnp.where(lo8, mu_v[0] * aw0, 0.0)),
              jnp.sum(jnp.where(lo8, z16, mu_v[0] * aw0)),
              jnp.sum(jnp.where(lo8, mu_v[1] * aw1, 0.0)),
              jnp.sum(jnp.where(lo8, z16, mu_v[1] * aw1))]

        hstats = []
        for q in range(2):
            awrot = [plsc.load_gather(
                awbuf, [q * _L + ((j + iota) & 15)]) for j in range(_L)]
            masks = [((j + iota) & 15) < 8 for j in range(_L)]
            crot = [q * _L + ((j + iota) & 15) for j in range(_L)]
            hini0 = spl(32 + 2 * q) - bc[2 * q]
            hini1 = spl(32 + 2 * q + 1) - bc[2 * q + 1]

            def ph_b(i, carry):
                hs0, hq0, hs1, hq1 = carry
                rid = i * _L + iota
                acc0 = z16
                acct = z16
                for j in range(_L):
                    xd = plsc.load_gather(xbuf, [rid, crot[j]])
                    p = xd * awrot[j]
                    acct = acct + p
                    acc0 = acc0 + jnp.where(masks[j], p, 0.0)
                h0 = jnp.maximum(acc0 + hini0, 0.0)
                h1 = jnp.maximum(acct - acc0 + hini1, 0.0)
                hbuf[pl.ds(2 * q * _B + i * _L, _L)] = h0
                hbuf[pl.ds((2 * q + 1) * _B + i * _L, _L)] = h1
                return (hs0 + h0, hq0 + h0 * h0, hs1 + h1, hq1 + h1 * h1)
            hs0, hq0, hs1, hq1 = lax.fori_loop(0, _B // _L, ph_b, (z16,) * 4)
            for hs, hq in ((hs0, hq0), (hs1, hq1)):
                m = jnp.sum(hs) * (1.0 / _B)
                var = jnp.sum(hq) * (1.0 / _B) - m * m
                hstats.append((m, _rsqrt_newton(var + _EPS)))

        a2 = []
        c2 = spl(40)
        for t in range(4):
            mh, rih = hstats[t]
            a2t = spl(36 + t) * rih
            a2.append(a2t)
            c2 = c2 - a2t * mh

        def ph_c(ii, carry):
            for u in range(2):
                i = ii * 2 + u
                zv = c2
                for t in range(4):
                    zv = zv + hbuf[pl.ds(t * _B + i * _L, _L)] * a2[t]
                zbuf[pl.ds(i * _L, _L)] = jnp.maximum(zv, 0.0)
            return carry
        lax.fori_loop(0, _B // _L // 2, ph_c, 0)

        for q in range(2):
            masks = [((j + iota) & 15) < 8 for j in range(_L)]
            crot = [q * _L + ((j + iota) & 15) for j in range(_L)]
            wgr = [wbuf[pl.ds(_OFF_WGR + (q * _L + j) * _L, _L)]
                   for j in range(_L)]
            bgr = [wbuf[pl.ds(_OFF_BGR + (q * _L + j) * _L, _L)]
                   for j in range(_L)]
            wdl0 = spl(41 + 2 * q)
            bdl0 = spl(45 + 2 * q)
            wdl1 = spl(41 + 2 * q + 1)
            bdl1 = spl(45 + 2 * q + 1)

            def ph_d(i, carry):
                rid = i * _L + iota
                zv = zbuf[pl.ds(i * _L, _L)]
                dl0 = jnp.maximum(zv * wdl0 + bdl0, 0.0)
                dl1 = jnp.maximum(zv * wdl1 + bdl1, 0.0)
                for j in range(_L):
                    val = jnp.where(masks[j], dl0, dl1) * wgr[j] + bgr[j]
                    plsc.store_scatter(xbuf, [rid, crot[j]], val)
                return carry
            lax.fori_loop(0, _B // _L, ph_d, 0)

        for j in range(16):
            pltpu.async_copy(xbuf.at[pl.ds(j * 128, 128)],
                             out_hbm.at[idxbuf.at[j]], sem)
        for j in range(16):
            pltpu.make_async_copy(xbuf.at[pl.ds(j * 128, 128)],
                                  out_hbm.at[idxbuf.at[j]], sem).wait()


def _sc_call(x, wtab):
    mesh = plsc.VectorSubcoreMesh(core_axis_name="c", subcore_axis_name="s")
    fn = functools.partial(
        pl.kernel,
        mesh=mesh,
        compiler_params=pltpu.CompilerParams(use_tc_tiling_on_sc=False,
                                             needs_layout_passes=False),
        out_type=jax.ShapeDtypeStruct((_B * _N_MID, _GPM), jnp.float32),
        scratch_types=[
            pltpu.VMEM((_B, _GPM), jnp.float32),
            pltpu.VMEM((4 * _B,), jnp.float32),
            pltpu.VMEM((_B,), jnp.float32),
            pltpu.VMEM((_WTAB,), jnp.float32),
            pltpu.VMEM((2 * _L,), jnp.float32),
            pltpu.VMEM((16, 128), jnp.int32),
            pltpu.SemaphoreType.DMA,
        ],
    )(_sc_body)
    out = fn(x.reshape(_B * _N_MID, _GPM), wtab.reshape(-1))
    return out.reshape(_B, _G)


def _make_wtab(W_enc_leaf, b_enc_leaf, W_enc_mid, b_enc_mid,
               w_dec_leaf, b_dec_leaf, w_dec_gene, b_dec_gene):
    f32 = jnp.float32
    scal = jnp.concatenate([
        W_enc_leaf.reshape(_N_MID, 32).astype(f32),
        b_enc_leaf.reshape(_N_MID, 4).astype(f32),
        W_enc_mid.reshape(_N_MID, 4).astype(f32),
        b_enc_mid.reshape(_N_MID, 1).astype(f32),
        w_dec_leaf.reshape(_N_MID, 4).astype(f32),
        b_dec_leaf.reshape(_N_MID, 4).astype(f32),
        w_dec_gene.reshape(_N_MID, 32).astype(f32),
        b_dec_gene.reshape(_N_MID, 32).astype(f32),
        jnp.zeros((_N_MID, 15), f32),
    ], axis=1)
    spl16 = jnp.repeat(scal[:, :, None], _L, axis=2).reshape(_N_MID, 128 * _L)
    qq, jj, ll = np.meshgrid(np.arange(2), np.arange(_L), np.arange(_L),
                             indexing="ij")
    rot = (qq * _L + (jj + ll) % _L).reshape(-1)
    wg = w_dec_gene.reshape(_N_MID, 32).astype(f32)[:, rot]
    bg = b_dec_gene.reshape(_N_MID, 32).astype(f32)[:, rot]
    wal = W_enc_leaf.reshape(_N_MID, 32).astype(f32)
    pad = jnp.zeros((_N_MID, _WTAB - 3104), f32)
    return jnp.concatenate([spl16, wg, bg, wal, pad], axis=1)


_BLK_G = 128
_BLK_LEAF = _BLK_G // _GPL
_BLK_MID = _BLK_LEAF // _LPM
_NBLK = _G // _BLK_G


def _tc_block_body(x_ref, wl_ref, bl_ref, wm_ref, bm_ref,
                   wdl_ref, bdl_ref, wdg_ref, bdg_ref, out_ref):
    xb = x_ref[...]
    mu = jnp.mean(xb, axis=0, keepdims=True)
    var = jnp.mean(xb * xb, axis=0, keepdims=True) - mu * mu
    xn = (xb - mu) * lax.rsqrt(var + _EPS)
    hp = jnp.dot(xn, wl_ref[0], preferred_element_type=jnp.float32)
    h = jnp.maximum(hp + bl_ref[0], 0.0)
    muh = jnp.mean(h, axis=0, keepdims=True)
    varh = jnp.mean(h * h, axis=0, keepdims=True) - muh * muh
    hn = (h - muh) * lax.rsqrt(varh + _EPS)
    zp = jnp.dot(hn, wm_ref[0], preferred_element_type=jnp.float32)
    z = jnp.maximum(zp + bm_ref[0], 0.0)
    e4 = (lax.broadcasted_iota(jnp.int32, (_BLK_MID, _BLK_LEAF), 1)
          // _LPM == lax.broadcasted_iota(
              jnp.int32, (_BLK_MID, _BLK_LEAF), 0)).astype(jnp.float32)
    zx = jnp.dot(z, e4, preferred_element_type=jnp.float32)
    dl = jnp.maximum(zx * wdl_ref[0] + bdl_ref[0], 0.0)
    e16 = (lax.broadcasted_iota(jnp.int32, (_BLK_LEAF, _BLK_G), 1)
           // _GPL == lax.broadcasted_iota(
               jnp.int32, (_BLK_LEAF, _BLK_G), 0)).astype(jnp.float32)
    dx = jnp.dot(dl, e16, preferred_element_type=jnp.float32)
    out_ref[...] = dx * wdg_ref[0] + bdg_ref[0]


def _tc_call(x, W_enc_leaf, b_enc_leaf, W_enc_mid, b_enc_mid,
             w_dec_leaf, b_dec_leaf, w_dec_gene, b_dec_gene):
    f32 = jnp.float32
    gl = np.arange(_BLK_G)
    tl = np.arange(_BLK_LEAF)
    leaf_mask = (gl[:, None] // _GPL == tl[None, :])
    w_leaf_b = W_enc_leaf.reshape(_NBLK, _BLK_LEAF, _GPL)
    wl = jnp.where(leaf_mask[None],
                   w_leaf_b.transpose(0, 2, 1)[:, gl % _GPL, :], 0.0)
    bl = b_enc_leaf.reshape(_NBLK, 1, _BLK_LEAF)
    mid_mask = (tl[:, None] // _LPM == np.arange(_BLK_MID)[None, :])
    w_mid_b = W_enc_mid.reshape(_NBLK, _BLK_MID, _LPM)
    wm = jnp.where(mid_mask[None],
                   w_mid_b.transpose(0, 2, 1)[:, tl % _LPM, :], 0.0)
    bm = b_enc_mid.reshape(_NBLK, 1, _BLK_MID)
    wdl = w_dec_leaf.reshape(_NBLK, 1, _BLK_LEAF)
    bdl = b_dec_leaf.reshape(_NBLK, 1, _BLK_LEAF)
    wdg = w_dec_gene.reshape(_NBLK, 1, _BLK_G)
    bdg = b_dec_gene.reshape(_NBLK, 1, _BLK_G)
    return pl.pallas_call(
        _tc_block_body,
        grid=(_NBLK,),
        in_specs=[
            pl.BlockSpec((_B, _BLK_G), lambda j: (0, j)),
            pl.BlockSpec((1, _BLK_G, _BLK_LEAF), lambda j: (j, 0, 0)),
            pl.BlockSpec((1, 1, _BLK_LEAF), lambda j: (j, 0, 0)),
            pl.BlockSpec((1, _BLK_LEAF, _BLK_MID), lambda j: (j, 0, 0)),
            pl.BlockSpec((1, 1, _BLK_MID), lambda j: (j, 0, 0)),
            pl.BlockSpec((1, 1, _BLK_LEAF), lambda j: (j, 0, 0)),
            pl.BlockSpec((1, 1, _BLK_LEAF), lambda j: (j, 0, 0)),
            pl.BlockSpec((1, 1, _BLK_G), lambda j: (j, 0, 0)),
            pl.BlockSpec((1, 1, _BLK_G), lambda j: (j, 0, 0)),
        ],
        out_specs=pl.BlockSpec((_B, _BLK_G), lambda j: (0, j)),
        out_shape=jax.ShapeDtypeStruct((_B, _G), f32),
    )(x, wl, bl, wm, bm, wdl, bdl, wdg, bdg)


def kernel(x, W_enc_leaf, b_enc_leaf, W_enc_mid, b_enc_mid,
           w_dec_mid, b_dec_mid, w_dec_leaf, b_dec_leaf,
           w_dec_gene, b_dec_gene):
    wtab = _make_wtab(W_enc_leaf, b_enc_leaf, W_enc_mid, b_enc_mid,
                      w_dec_leaf, b_dec_leaf, w_dec_gene, b_dec_gene)
    return _sc_call(x, wtab)


# speedup vs baseline: 1.1467x; 1.1467x over previous
import functools

import jax
import jax.numpy as jnp
import numpy as np
from jax import lax
from jax.experimental import pallas as pl
from jax.experimental.pallas import tpu as pltpu
from jax.experimental.pallas import tpu_sc as plsc

_B = 2048
_G = 2048
_N_LEAF = 256
_GPL = 8
_N_MID = 64
_LPM = 4
_GPM = _GPL * _LPM
_EPS = 1e-5


_NW = 32
_MPW = 1
_L = 16
_OFF_WGR = 2048
_OFF_BGR = 2560
_OFF_WAL = 3072
_WTAB = 3136


def _rsqrt_newton(v):
    i = lax.bitcast_convert_type(v, jnp.int32)
    y = lax.bitcast_convert_type(jnp.int32(0x5F3759DF) - (i >> 1), jnp.float32)
    for _ in range(4):
        y = y * (1.5 - 0.5 * v * y * y)
    return y


def _sc_body(x_hbm, wtab_hbm, out_hbm, xbuf, hbuf, zbuf, wbuf, awbuf,
             idxbuf, sem):
    f32 = jnp.float32
    i32 = jnp.int32
    iota = lax.broadcasted_iota(i32, (_L,), 0)
    z16 = jnp.zeros((_L,), f32)

    def spl(off):
        return wbuf[pl.ds(off * _L, _L)]

    wid = lax.axis_index("s") * 2 + lax.axis_index("c")
    for mm in range(_MPW):
        mid = wid * _MPW + mm
        pltpu.sync_copy(wtab_hbm.at[pl.ds(mid * _WTAB, _WTAB)], wbuf)

        for j in range(16):
            for u in range(8):
                idxbuf[j, pl.ds(u * _L, _L)] = (
                    (j * 128 + u * _L + iota) * _N_MID + mid)
        for j in range(16):
            pltpu.async_copy(x_hbm.at[idxbuf.at[j]],
                             xbuf.at[pl.ds(j * 128, 128)], sem)
        for j in range(16):
            pltpu.make_async_copy(x_hbm.at[idxbuf.at[j]],
                                  xbuf.at[pl.ds(j * 128, 128)], sem).wait()

        def ph_a(ii, carry):
            acc = list(carry)
            for u in range(4):
                row = jnp.full((_L,), ii * 4 + u, i32)
                v0 = plsc.load_gather(xbuf, [row, iota])
                v1 = plsc.load_gather(xbuf, [row, iota + _L])
                acc[0] = acc[0] + v0
                acc[1] = acc[1] + v1
                acc[2] = acc[2] + v0 * v0
                acc[3] = acc[3] + v1 * v1
            return tuple(acc)
        acc = lax.fori_loop(0, _B // 4, ph_a, (z16,) * 4)
        mu_v = [acc[0] * (1.0 / _B), acc[1] * (1.0 / _B)]
        var_v = [acc[2] * (1.0 / _B) - mu_v[0] * mu_v[0],
                 acc[3] * (1.0 / _B) - mu_v[1] * mu_v[1]]
        rinv_v = [_rsqrt_newton(var_v[0] + _EPS),
                  _rsqrt_newton(var_v[1] + _EPS)]
        aw0 = wbuf[pl.ds(_OFF_WAL, _L)] * rinv_v[0]
        aw1 = wbuf[pl.ds(_OFF_WAL + _L, _L)] * rinv_v[1]
        awbuf[pl.ds(0, _L)] = aw0
        awbuf[pl.ds(_L, _L)] = aw1
        lo8 = (iota & 15) < 8
        bc = [jnp.sum(jnp.where(lo8, mu_v[0] * aw0, 0.0)),
              jnp.sum(jnp.where(lo8, z16, mu_v[0] * aw0)),
              jnp.sum(jnp.where(lo8, mu_v[1] * aw1, 0.0)),
              jnp.sum(jnp.where(lo8, z16, mu_v[1] * aw1))]

        hstats = []
        for q in range(2):
            awrot = [plsc.load_gather(
                awbuf, [q * _L + ((j + iota) & 15)]) for j in range(_L)]
            masks = [((j + iota) & 15) < 8 for j in range(_L)]
            crot = [q * _L + ((j + iota) & 15) for j in range(_L)]
            hini0 = spl(32 + 2 * q) - bc[2 * q]
            hini1 = spl(32 + 2 * q + 1) - bc[2 * q + 1]

            def ph_b(i, carry):
                hs0, hq0, hs1, hq1 = carry
                rid = i * _L + iota
                acc0 = z16
                acct = z16
                for j in range(_L):
                    xd = plsc.load_gather(xbuf, [rid, crot[j]])
                    p = xd * awrot[j]
                    acct = acct + p
                    acc0 = acc0 + jnp.where(masks[j], p, 0.0)
                h0 = jnp.maximum(acc0 + hini0, 0.0)
                h1 = jnp.maximum(acct - acc0 + hini1, 0.0)
                hbuf[pl.ds(2 * q * _B + i * _L, _L)] = h0
                hbuf[pl.ds((2 * q + 1) * _B + i * _L, _L)] = h1
                return (hs0 + h0, hq0 + h0 * h0, hs1 + h1, hq1 + h1 * h1)
            hs0, hq0, hs1, hq1 = lax.fori_loop(0, _B // _L, ph_b, (z16,) * 4)
            for hs, hq in ((hs0, hq0), (hs1, hq1)):
                m = jnp.sum(hs) * (1.0 / _B)
                var = jnp.sum(hq) * (1.0 / _B) - m * m
                hstats.append((m, _rsqrt_newton(var + _EPS)))

        a2 = []
        c2 = spl(40)
        for t in range(4):
            mh, rih = hstats[t]
            a2t = spl(36 + t) * rih
            a2.append(a2t)
            c2 = c2 - a2t * mh

        def ph_c(ii, carry):
            for u in range(2):
                i = ii * 2 + u
                zv = c2
                for t in range(4):
                    zv = zv + hbuf[pl.ds(t * _B + i * _L, _L)] * a2[t]
                zbuf[pl.ds(i * _L, _L)] = jnp.maximum(zv, 0.0)
            return carry
        lax.fori_loop(0, _B // _L // 2, ph_c, 0)

        for q in range(2):
            masks = [((j + iota) & 15) < 8 for j in range(_L)]
            crot = [q * _L + ((j + iota) & 15) for j in range(_L)]
            wgr = [wbuf[pl.ds(_OFF_WGR + (q * _L + j) * _L, _L)]
                   for j in range(_L)]
            bgr = [wbuf[pl.ds(_OFF_BGR + (q * _L + j) * _L, _L)]
                   for j in range(_L)]
            wdl0 = spl(41 + 2 * q)
            bdl0 = spl(45 + 2 * q)
            wdl1 = spl(41 + 2 * q + 1)
            bdl1 = spl(45 + 2 * q + 1)

            def ph_d(i, carry):
                rid = i * _L + iota
                zv = zbuf[pl.ds(i * _L, _L)]
                dl0 = jnp.maximum(zv * wdl0 + bdl0, 0.0)
                dl1 = jnp.maximum(zv * wdl1 + bdl1, 0.0)
                for j in range(_L):
                    val = jnp.where(masks[j], dl0, dl1) * wgr[j] + bgr[j]
                    plsc.store_scatter(xbuf, [rid, crot[j]], val)
                return carry
            lax.fori_loop(0, _B // _L, ph_d, 0)

        for j in range(16):
            pltpu.async_copy(xbuf.at[pl.ds(j * 128, 128)],
                             out_hbm.at[idxbuf.at[j]], sem)
        for j in range(16):
            pltpu.make_async_copy(xbuf.at[pl.ds(j * 128, 128)],
                                  out_hbm.at[idxbuf.at[j]], sem).wait()


def _sc_call(x, wtab):
    mesh = plsc.VectorSubcoreMesh(core_axis_name="c", subcore_axis_name="s")
    fn = functools.partial(
        pl.kernel,
        mesh=mesh,
        compiler_params=pltpu.CompilerParams(use_tc_tiling_on_sc=False,
                                             needs_layout_passes=False),
        out_type=jax.ShapeDtypeStruct((_B * _N_MID, _GPM), jnp.float32),
        scratch_types=[
            pltpu.VMEM((_B, _GPM), jnp.float32),
            pltpu.VMEM((4 * _B,), jnp.float32),
            pltpu.VMEM((_B,), jnp.float32),
            pltpu.VMEM((_WTAB,), jnp.float32),
            pltpu.VMEM((2 * _L,), jnp.float32),
            pltpu.VMEM((16, 128), jnp.int32),
            pltpu.SemaphoreType.DMA,
        ],
    )(_sc_body)
    out = fn(x.reshape(_B * _N_MID, _GPM), wtab.reshape(-1))
    return out.reshape(_B, _G)


def _make_wtab(W_enc_leaf, b_enc_leaf, W_enc_mid, b_enc_mid,
               w_dec_leaf, b_dec_leaf, w_dec_gene, b_dec_gene):
    f32 = jnp.float32
    scal = jnp.concatenate([
        W_enc_leaf.reshape(_N_MID, 32).astype(f32),
        b_enc_leaf.reshape(_N_MID, 4).astype(f32),
        W_enc_mid.reshape(_N_MID, 4).astype(f32),
        b_enc_mid.reshape(_N_MID, 1).astype(f32),
        w_dec_leaf.reshape(_N_MID, 4).astype(f32),
        b_dec_leaf.reshape(_N_MID, 4).astype(f32),
        w_dec_gene.reshape(_N_MID, 32).astype(f32),
        b_dec_gene.reshape(_N_MID, 32).astype(f32),
        jnp.zeros((_N_MID, 15), f32),
    ], axis=1)
    spl16 = jnp.repeat(scal[:, :, None], _L, axis=2).reshape(_N_MID, 128 * _L)
    qq, jj, ll = np.meshgrid(np.arange(2), np.arange(_L), np.arange(_L),
                             indexing="ij")
    rot = (qq * _L + (jj + ll) % _L).reshape(-1)
    wg = w_dec_gene.reshape(_N_MID, 32).astype(f32)[:, rot]
    bg = b_dec_gene.reshape(_N_MID, 32).astype(f32)[:, rot]
    wal = W_enc_leaf.reshape(_N_MID, 32).astype(f32)
    pad = jnp.zeros((_N_MID, _WTAB - 3104), f32)
    return jnp.concatenate([spl16, wg, bg, wal, pad], axis=1)


_BLK_G = 128
_BLK_LEAF = _BLK_G // _GPL
_BLK_MID = _BLK_LEAF // _LPM
_NBLK = _G // _BLK_G


def _tc_block_body(x_ref, wl_ref, bl_ref, wm_ref, bm_ref,
                   wdl_ref, bdl_ref, wdg_ref, bdg_ref, out_ref):
    xb = x_ref[...]
    mu = jnp.mean(xb, axis=0, keepdims=True)
    var = jnp.mean(xb * xb, axis=0, keepdims=True) - mu * mu
    xn = (xb - mu) * lax.rsqrt(var + _EPS)
    hp = jnp.dot(xn, wl_ref[0], preferred_element_type=jnp.float32)
    h = jnp.maximum(hp + bl_ref[0], 0.0)
    muh = jnp.mean(h, axis=0, keepdims=True)
    varh = jnp.mean(h * h, axis=0, keepdims=True) - muh * muh
    hn = (h - muh) * lax.rsqrt(varh + _EPS)
    zp = jnp.dot(hn, wm_ref[0], preferred_element_type=jnp.float32)
    z = jnp.maximum(zp + bm_ref[0], 0.0)
    e4 = (lax.broadcasted_iota(jnp.int32, (_BLK_MID, _BLK_LEAF), 1)
          // _LPM == lax.broadcasted_iota(
              jnp.int32, (_BLK_MID, _BLK_LEAF), 0)).astype(jnp.float32)
    zx = jnp.dot(z, e4, preferred_element_type=jnp.float32)
    dl = jnp.maximum(zx * wdl_ref[0] + bdl_ref[0], 0.0)
    e16 = (lax.broadcasted_iota(jnp.int32, (_BLK_LEAF, _BLK_G), 1)
           // _GPL == lax.broadcasted_iota(
               jnp.int32, (_BLK_LEAF, _BLK_G), 0)).astype(jnp.float32)
    dx = jnp.dot(dl, e16, preferred_element_type=jnp.float32)
    out_ref[...] = dx * wdg_ref[0] + bdg_ref[0]


def _tc_call(x, W_enc_leaf, b_enc_leaf, W_enc_mid, b_enc_mid,
             w_dec_leaf, b_dec_leaf, w_dec_gene, b_dec_gene, nblk=_NBLK):
    f32 = jnp.float32
    gl = np.arange(_BLK_G)
    tl = np.arange(_BLK_LEAF)
    leaf_mask = (gl[:, None] // _GPL == tl[None, :])
    w_leaf_b = W_enc_leaf.reshape(_NBLK, _BLK_LEAF, _GPL)
    wl = jnp.where(leaf_mask[None],
                   w_leaf_b.transpose(0, 2, 1)[:, gl % _GPL, :], 0.0)
    bl = b_enc_leaf.reshape(_NBLK, 1, _BLK_LEAF)
    mid_mask = (tl[:, None] // _LPM == np.arange(_BLK_MID)[None, :])
    w_mid_b = W_enc_mid.reshape(_NBLK, _BLK_MID, _LPM)
    wm = jnp.where(mid_mask[None],
                   w_mid_b.transpose(0, 2, 1)[:, tl % _LPM, :], 0.0)
    bm = b_enc_mid.reshape(_NBLK, 1, _BLK_MID)
    wdl = w_dec_leaf.reshape(_NBLK, 1, _BLK_LEAF)
    bdl = b_dec_leaf.reshape(_NBLK, 1, _BLK_LEAF)
    wdg = w_dec_gene.reshape(_NBLK, 1, _BLK_G)
    bdg = b_dec_gene.reshape(_NBLK, 1, _BLK_G)
    blk0 = _NBLK - nblk
    return pl.pallas_call(
        _tc_block_body,
        grid=(nblk,),
        in_specs=[
            pl.BlockSpec((_B, _BLK_G), lambda j: (0, j + blk0)),
            pl.BlockSpec((1, _BLK_G, _BLK_LEAF), lambda j: (j + blk0, 0, 0)),
            pl.BlockSpec((1, 1, _BLK_LEAF), lambda j: (j + blk0, 0, 0)),
            pl.BlockSpec((1, _BLK_LEAF, _BLK_MID), lambda j: (j + blk0, 0, 0)),
            pl.BlockSpec((1, 1, _BLK_MID), lambda j: (j + blk0, 0, 0)),
            pl.BlockSpec((1, 1, _BLK_LEAF), lambda j: (j + blk0, 0, 0)),
            pl.BlockSpec((1, 1, _BLK_LEAF), lambda j: (j + blk0, 0, 0)),
            pl.BlockSpec((1, 1, _BLK_G), lambda j: (j + blk0, 0, 0)),
            pl.BlockSpec((1, 1, _BLK_G), lambda j: (j + blk0, 0, 0)),
        ],
        out_specs=pl.BlockSpec((_B, _BLK_G), lambda j: (0, j + blk0)),
        out_shape=jax.ShapeDtypeStruct((_B, _G), f32),
    )(x, wl, bl, wm, bm, wdl, bdl, wdg, bdg)


_SC_MIDS = _NW * _MPW


def kernel(x, W_enc_leaf, b_enc_leaf, W_enc_mid, b_enc_mid,
           w_dec_mid, b_dec_mid, w_dec_leaf, b_dec_leaf,
           w_dec_gene, b_dec_gene):
    wtab = _make_wtab(W_enc_leaf, b_enc_leaf, W_enc_mid, b_enc_mid,
                      w_dec_leaf, b_dec_leaf, w_dec_gene, b_dec_gene)
    sc = _sc_call(x, wtab)
    csplit = _SC_MIDS * _GPM
    if csplit == _G:
        return sc
    tc = _tc_call(x, W_enc_leaf, b_enc_leaf, W_enc_mid, b_enc_mid,
                  w_dec_leaf, b_dec_leaf, w_dec_gene, b_dec_gene,
                  nblk=(_G - csplit) // _BLK_G)
    return jnp.concatenate([sc[:, :csplit], tc[:, csplit:]], axis=1)
